# baseline (speedup 1.0000x reference)
import functools

import jax
import jax.numpy as jnp
from jax import lax
from jax.experimental import pallas as pl
from jax.experimental.pallas import tpu as pltpu
from jax.experimental.pallas import tpu_sc as plsc

VOCAB = 1000000
DIM = 64
BATCH = 4096
HIST = 50

_info = plsc.get_sparse_core_info()
_NC = _info.num_cores
_NS = _info.num_subcores
NW = _NC * _NS
BPW = BATCH // NW
CB = 16
NBG = BPW // CB
HH = 8
NHG = HIST // HH
HTAIL = HIST - NHG * HH

_mesh = plsc.VectorSubcoreMesh(core_axis_name="c", subcore_axis_name="s")


@functools.partial(
    pl.kernel,
    mesh=_mesh,
    out_type=jax.ShapeDtypeStruct((BATCH, HIST, DIM), jnp.float32),
    scratch_types=[
        pltpu.VMEM((HIST, BPW), jnp.int32),
        pltpu.VMEM((2, HH, CB), jnp.int32),
        pltpu.VMEM((2, HH, CB, 128), jnp.float32),
        pltpu.VMEM((2, CB, HH, DIM), jnp.float32),
        pltpu.SemaphoreType.DMA,
        pltpu.SemaphoreType.DMA,
        pltpu.SemaphoreType.DMA,
        pltpu.SemaphoreType.DMA,
    ],
)
def _emb_lookup(idsT_hbm, t2_hbm, out_hbm, idx_v, pair_v, rbuf, sbuf,
                g0, g1, o0, o1):
    wid = lax.axis_index("s") * _NC + lax.axis_index("c")
    b0 = wid * BPW
    pltpu.sync_copy(idsT_hbm.at[:, pl.ds(b0, BPW)], idx_v)

    def issue(bg, h0, hh, slot):
        gsem = g0 if slot == 0 else g1

        def step(h, carry):
            v = idx_v[h0 + h, pl.ds(bg * CB, CB)]
            pair_v[slot, h, pl.ds(0, CB)] = lax.shift_right_logical(v, 1)
            pltpu.async_copy(
                t2_hbm.at[pair_v.at[slot, h]], rbuf.at[slot, h], gsem
            )
            return carry

        lax.fori_loop(0, hh, step, 0)

    def drain(hh, slot):
        gsem = g0 if slot == 0 else g1

        def step(h, carry):
            pltpu.make_async_copy(
                t2_hbm.at[pl.ds(0, CB)], rbuf.at[slot, h], gsem
            ).wait()
            return carry

        lax.fori_loop(0, hh, step, 0)

    def select(bg, h0, hh, slot):
        def step(h, carry):
            v = idx_v[h0 + h, pl.ds(bg * CB, CB)]
            for u in range(CB):
                off = (v[u] & 1) * DIM
                for q in range(DIM // 16):
                    sbuf[slot, u, h, pl.ds(q * 16, 16)] = (
                        rbuf[slot, h, u, pl.ds(off + q * 16, 16)]
                    )
            return carry

        lax.fori_loop(0, hh, step, 0)

    def write(bg, h0, hh, slot):
        osem = o0 if slot == 0 else o1
        pltpu.async_copy(
            sbuf.at[slot, :, pl.ds(0, hh)],
            out_hbm.at[pl.ds(b0 + bg * CB, CB), pl.ds(h0, hh)],
            osem,
        )

    def wait_write(hh, slot):
        osem = o0 if slot == 0 else o1
        pltpu.make_async_copy(
            sbuf.at[slot, :, pl.ds(0, hh)],
            out_hbm.at[pl.ds(b0, CB), pl.ds(0, hh)],
            osem,
        ).wait()

    chunks = [(g * HH, HH) for g in range(NHG)] + [(NHG * HH, HTAIL)]

    def bg_body(bg, carry):
        issue(bg, chunks[0][0], chunks[0][1], 0)
        prev = [None, None]
        for i, (h0, hh) in enumerate(chunks):
            slot = i % 2
            nslot = (i + 1) % 2
            if i + 1 < len(chunks):
                nh0, nhh = chunks[i + 1]
                if prev[nslot] is not None:
                    wait_write(prev[nslot], nslot)
                issue(bg, nh0, nhh, nslot)
            drain(hh, slot)
            select(bg, h0, hh, slot)
            write(bg, h0, hh, slot)
            prev[slot] = hh
        wait_write(prev[0], 0)
        wait_write(prev[1], 1)
        return carry

    lax.fori_loop(0, NBG, bg_body, 0)


def kernel(input_ids, table):
    t2 = table.reshape(VOCAB // 2, 128)
    return _emb_lookup(input_ids.T, t2)


# speedup vs baseline: 2.1355x; 2.1355x over previous
import functools

import jax
import jax.numpy as jnp
from jax import lax
from jax.experimental import pallas as pl
from jax.experimental.pallas import tpu as pltpu
from jax.experimental.pallas import tpu_sc as plsc

VOCAB = 1000000
DIM = 64
BATCH = 4096
HIST = 50
N = BATCH * HIST

_info = plsc.get_sparse_core_info()
_NC = _info.num_cores
_NS = _info.num_subcores
NW = _NC * _NS
B_PER_W = BATCH // NW
ROWS_PER_W = B_PER_W * HIST
CB = 8
CR = CB * HIST
NCHUNK = B_PER_W // CB

_mesh = plsc.VectorSubcoreMesh(core_axis_name="c", subcore_axis_name="s")


@functools.partial(
    pl.kernel,
    mesh=_mesh,
    out_type=jax.ShapeDtypeStruct((BATCH, HIST, DIM), jnp.float32),
    scratch_types=[
        pltpu.VMEM((ROWS_PER_W,), jnp.int32),
        pltpu.VMEM((2, CB, HIST, DIM), jnp.float32),
        pltpu.SemaphoreType.DMA,
        pltpu.SemaphoreType.DMA,
        pltpu.SemaphoreType.DMA,
        pltpu.SemaphoreType.DMA,
    ],
)
def _emb_lookup(ids_hbm, table_hbm, out_hbm, idx_v, buf, g0, g1, o0, o1):
    wid = lax.axis_index("s") * _NC + lax.axis_index("c")
    base_row = wid * ROWS_PER_W
    base_batch = wid * B_PER_W
    pltpu.sync_copy(ids_hbm.at[pl.ds(base_row, ROWS_PER_W)], idx_v)
    gsems = (g0, g1)
    osems = (o0, o1)

    def issue_chunk(g, slot):
        def body(t, carry):
            v = idx_v[pl.ds(g * CR + t * 16, 16)]
            for u in range(16):
                j = t * 16 + u
                pltpu.async_copy(
                    table_hbm.at[
                        lax.shift_right_logical(v[u], 3), v[u] & 7
                    ],
                    buf.at[slot, j // HIST, j % HIST],
                    gsems[slot],
                )
            return carry
        lax.fori_loop(0, CR // 16, body, 0)

    def drain_chunk(slot):
        pltpu.make_async_copy(
            out_hbm.at[pl.ds(0, CB)], buf.at[slot], gsems[slot]
        ).wait()

    def write_chunk(g, slot):
        return pltpu.async_copy(
            buf.at[slot], out_hbm.at[pl.ds(base_batch + g * CB, CB)], osems[slot]
        )

    def wait_write(slot):
        pltpu.make_async_copy(
            buf.at[slot], out_hbm.at[pl.ds(base_batch, CB)], osems[slot]
        ).wait()

    issue_chunk(0, 0)
    drain_chunk(0)
    write_chunk(0, 0)
    for g in range(1, NCHUNK):
        slot = g % 2
        if g >= 2:
            wait_write(slot)
        issue_chunk(g, slot)
        drain_chunk(slot)
        write_chunk(g, slot)
    wait_write(0)
    wait_write(1)


def kernel(input_ids, table):
    ids = input_ids.reshape(-1).astype(jnp.int32)
    t3 = table.reshape(VOCAB // 8, 8, DIM)
    return _emb_lookup(ids, t3)


# speedup vs baseline: 2.1360x; 1.0002x over previous
import functools

import jax
import jax.numpy as jnp
from jax import lax
from jax.experimental import pallas as pl
from jax.experimental.pallas import tpu as pltpu
from jax.experimental.pallas import tpu_sc as plsc

VOCAB = 1000000
DIM = 64
BATCH = 4096
HIST = 50
N = BATCH * HIST

_info = plsc.get_sparse_core_info()
_NC = _info.num_cores
_NS = _info.num_subcores
NW = _NC * _NS
B_PER_W = BATCH // NW
ROWS_PER_W = B_PER_W * HIST
CB = 8
CR = CB * HIST
NCHUNK = B_PER_W // CB

_mesh = plsc.VectorSubcoreMesh(core_axis_name="c", subcore_axis_name="s")


@functools.partial(
    pl.kernel,
    mesh=_mesh,
    out_type=jax.ShapeDtypeStruct((BATCH, HIST, DIM), jnp.float32),
    scratch_types=[
        pltpu.VMEM((ROWS_PER_W,), jnp.int32),
        pltpu.VMEM((2, CR, DIM), jnp.float32),
        pltpu.SemaphoreType.DMA,
        pltpu.SemaphoreType.DMA,
        pltpu.SemaphoreType.DMA,
        pltpu.SemaphoreType.DMA,
    ],
)
def _emb_lookup(ids_hbm, table_hbm, out_hbm, idx_v, buf, g0, g1, o0, o1):
    wid = lax.axis_index("s") * _NC + lax.axis_index("c")
    base_row = wid * ROWS_PER_W
    base_batch = wid * B_PER_W
    pltpu.sync_copy(ids_hbm.at[pl.ds(base_row, ROWS_PER_W)], idx_v)
    gsems = (g0, g1)
    osems = (o0, o1)

    def issue_chunk(g, slot):
        def body(t, carry):
            v = idx_v[pl.ds(g * CR + t * 16, 16)]
            for u in range(16):
                j = t * 16 + u
                pltpu.async_copy(
                    table_hbm.at[
                        lax.shift_right_logical(v[u], 3), v[u] & 7
                    ],
                    buf.at[slot, j],
                    gsems[slot],
                )
            return carry
        lax.fori_loop(0, CR // 16, body, 0)

    def drain_chunk(slot):
        pltpu.make_async_copy(
            out_hbm.at[0], buf.at[slot, pl.ds(0, HIST)], gsems[slot]
        ).wait()

        def extra(b, carry):
            pltpu.make_async_copy(
                out_hbm.at[0], buf.at[slot, pl.ds(0, HIST)], gsems[slot]
            ).wait()
            return carry

        lax.fori_loop(0, CB - 1, extra, 0)

    def write_chunk(g, slot):
        for b in range(CB):
            pltpu.async_copy(
                buf.at[slot, pl.ds(b * HIST, HIST)],
                out_hbm.at[base_batch + g * CB + b],
                osems[slot],
            )

    def wait_write(slot):
        def step(b, carry):
            pltpu.make_async_copy(
                buf.at[slot, pl.ds(0, HIST)], out_hbm.at[0], osems[slot]
            ).wait()
            return carry

        lax.fori_loop(0, CB, step, 0)

    issue_chunk(0, 0)
    drain_chunk(0)
    write_chunk(0, 0)
    for g in range(1, NCHUNK):
        slot = g % 2
        if g >= 2:
            wait_write(slot)
        issue_chunk(g, slot)
        drain_chunk(slot)
        write_chunk(g, slot)
    wait_write(0)
    wait_write(1)


def kernel(input_ids, table):
    ids = input_ids.reshape(-1).astype(jnp.int32)
    t3 = table.reshape(VOCAB // 8, 8, DIM)
    return _emb_lookup(ids, t3)
